# Initial kernel scaffold; baseline (speedup 1.0000x reference)
import jax
import jax.numpy as jnp
from jax.experimental import pallas as pl


def kernel(grid, guide, image):
    raise NotImplementedError("write your pallas kernel here")


# speedup vs baseline: 203.1826x; 203.1826x over previous
import functools

import jax
import jax.numpy as jnp
import numpy as np
from jax import lax
from jax.experimental import pallas as pl
from jax.experimental.pallas import tpu as pltpu
from jax.experimental.pallas import tpu_sc as plsc

_B, _C, _GD, _GH, _GW = 4, 12, 8, 16, 16
_H = _W = 512
_NIN = 3
_NOUT = 3
_NW = 32
_RPW = _H // _NW
_RPC = 8
_K = _C * _GD * _GW


def _axis_tables(n_pix, n_grid):
    g = (np.arange(n_pix, dtype=np.float64) + 0.5) * n_grid / n_pix
    f = np.floor(g - 0.5)
    i0 = np.clip(f, 0, n_grid - 1).astype(np.int32)
    i1 = np.clip(f + 1, 0, n_grid - 1).astype(np.int32)
    d = (g - (f + 0.5)).astype(np.float32)
    return i0, i1, d


def _build_my():
    y0, y1, dy = _axis_tables(_H, _GH)
    my = np.zeros((_H, _GH), dtype=np.float32)
    np.add.at(my, (np.arange(_H), y0), 1.0 - dy)
    np.add.at(my, (np.arange(_H), y1), dy)
    return my


def _tc_y_upsample(my, gt):
    def body(my_ref, g_ref, o_ref):
        o_ref[0] = jnp.dot(my_ref[...], g_ref[0],
                           preferred_element_type=jnp.float32,
                           precision=jax.lax.Precision.HIGHEST)

    return pl.pallas_call(
        body,
        grid=(_B,),
        in_specs=[
            pl.BlockSpec((_H, _GH), lambda b: (0, 0)),
            pl.BlockSpec((1, _GH, _K), lambda b: (b, 0, 0)),
        ],
        out_specs=pl.BlockSpec((1, _H, _K), lambda b: (b, 0, 0)),
        out_shape=jax.ShapeDtypeStruct((_B, _H, _K), jnp.float32),
    )(my, gt)


def _make_sc_kernel():
    mesh = plsc.VectorSubcoreMesh(core_axis_name="c", subcore_axis_name="s",
                                  num_cores=2, num_subcores=16)
    chunk_px = _RPC * _W

    @functools.partial(
        pl.kernel,
        out_type=jax.ShapeDtypeStruct((_B * _NOUT, _H * _W), jnp.float32),
        mesh=mesh,
        compiler_params=pltpu.CompilerParams(needs_layout_passes=False),
        scratch_types=[
            pltpu.VMEM((_RPC * _K,), jnp.float32),
            pltpu.VMEM((chunk_px,), jnp.float32),
            pltpu.VMEM((_NIN * chunk_px,), jnp.float32),
            pltpu.VMEM((_NOUT * chunk_px,), jnp.float32),
            pltpu.VMEM((_W,), jnp.int32),
            pltpu.VMEM((_W,), jnp.int32),
            pltpu.VMEM((_W,), jnp.float32),
        ],
    )
    def sc_apply(t_hbm, guide_hbm, image_hbm, x0_hbm, x1_hbm, dx_hbm,
                 out_hbm, t_buf, g_buf, i_buf, o_buf, x0_buf, x1_buf,
                 dx_buf):
        wid = lax.axis_index("s") * 2 + lax.axis_index("c")
        pltpu.sync_copy(x0_hbm, x0_buf)
        pltpu.sync_copy(x1_hbm, x1_buf)
        pltpu.sync_copy(dx_hbm, dx_buf)

        def chunk_body(chunk, carry):
            b = chunk // 2
            h0 = wid * _RPW + (chunk % 2) * _RPC
            pltpu.sync_copy(t_hbm.at[b, pl.ds(h0 * _K, _RPC * _K)], t_buf)
            pltpu.sync_copy(guide_hbm.at[b, pl.ds(h0 * _W, chunk_px)], g_buf)
            for i in range(_NIN):
                pltpu.sync_copy(
                    image_hbm.at[b * _NIN + i, pl.ds(h0 * _W, chunk_px)],
                    i_buf.at[pl.ds(i * chunk_px, chunk_px)])

            for r in range(_RPC):
                rbase = r * _K

                def vec_body(v, c2, r=r, rbase=rbase):
                    off = r * _W + v * 16
                    g = g_buf[pl.ds(off, 16)]
                    tz = g * np.float32(_GD) + np.float32(_GD - 0.5)
                    fzi = tz.astype(jnp.int32)
                    dz = tz - fzi.astype(jnp.float32)
                    z0 = jnp.clip(fzi - _GD, 0, _GD - 1)
                    z1 = jnp.clip(fzi - (_GD - 1), 0, _GD - 1)
                    x0 = x0_buf[pl.ds(v * 16, 16)]
                    x1 = x1_buf[pl.ds(v * 16, 16)]
                    dxv = dx_buf[pl.ds(v * 16, 16)]
                    zb0 = z0 * _GW
                    zb1 = z1 * _GW
                    i00 = zb0 + x0
                    i01 = zb0 + x1
                    i10 = zb1 + x0
                    i11 = zb1 + x1
                    ims = [i_buf[pl.ds(i * chunk_px + off, 16)]
                           for i in range(_NIN)]
                    for o in range(_NOUT):
                        acc = None
                        for j in range(_NIN + 1):
                            cb = rbase + (o * (_NIN + 1) + j) * (_GD * _GW)
                            a00 = plsc.load_gather(t_buf, [i00 + cb])
                            a01 = plsc.load_gather(t_buf, [i01 + cb])
                            a10 = plsc.load_gather(t_buf, [i10 + cb])
                            a11 = plsc.load_gather(t_buf, [i11 + cb])
                            a0 = a00 + dxv * (a01 - a00)
                            a1 = a10 + dxv * (a11 - a10)
                            cf = a0 + dz * (a1 - a0)
                            term = cf * ims[j] if j < _NIN else cf
                            acc = term if acc is None else acc + term
                        o_buf[pl.ds(o * chunk_px + off, 16)] = acc
                    return c2

                lax.fori_loop(0, _W // 16, vec_body, 0)

            for o in range(_NOUT):
                pltpu.sync_copy(
                    o_buf.at[pl.ds(o * chunk_px, chunk_px)],
                    out_hbm.at[b * _NOUT + o, pl.ds(h0 * _W, chunk_px)])
            return carry

        lax.fori_loop(0, _B * (_RPW // _RPC), chunk_body, 0)

    return sc_apply


_SC_APPLY = functools.lru_cache(maxsize=None)(_make_sc_kernel)
_MY = _build_my()
_X0, _X1, _DX = _axis_tables(_W, _GW)


@jax.jit
def kernel(grid, guide, image):
    gt = jnp.transpose(grid, (0, 3, 1, 2, 4)).reshape(_B, _GH, _K)
    t_all = _tc_y_upsample(jnp.asarray(_MY), gt)
    out = _SC_APPLY()(
        t_all.reshape(_B, _H * _K),
        guide.reshape(_B, _H * _W),
        image.reshape(_B * _NIN, _H * _W),
        jnp.asarray(_X0), jnp.asarray(_X1), jnp.asarray(_DX),
    )
    return out.reshape(_B, _NOUT, _H, _W)


# speedup vs baseline: 425.8262x; 2.0958x over previous
import functools

import jax
import jax.numpy as jnp
import numpy as np
from jax import lax
from jax.experimental import pallas as pl
from jax.experimental.pallas import tpu as pltpu
from jax.experimental.pallas import tpu_sc as plsc

_B, _C, _GD, _GH, _GW = 4, 12, 8, 16, 16
_H = _W = 512
_NIN = 3
_NOUT = 3
_NW = 32
_RPW = _H // _NW
_RPC = 8
_GSZ = _C * _GD * _GH * _GW
_ZS = 19
_CS = _GD * _ZS
_TSZ = _C * _CS


def _axis_tables(n_pix, n_grid):
    g = (np.arange(n_pix, dtype=np.float64) + 0.5) * n_grid / n_pix
    f = np.floor(g - 0.5)
    i0 = np.clip(f, 0, n_grid - 1).astype(np.int32)
    i1 = np.clip(f + 1, 0, n_grid - 1).astype(np.int32)
    d = (g - (f + 0.5)).astype(np.float32)
    return i0, i1, d


def _make_sc_kernel():
    mesh = plsc.VectorSubcoreMesh(core_axis_name="c", subcore_axis_name="s",
                                  num_cores=2, num_subcores=16)
    chunk_px = _RPC * _W
    n_chunks = _B * (_RPW // _RPC)

    @functools.partial(
        pl.kernel,
        out_type=jax.ShapeDtypeStruct((_B * _NOUT, _H * _W), jnp.float32),
        mesh=mesh,
        compiler_params=pltpu.CompilerParams(needs_layout_passes=False),
        scratch_types=[
            pltpu.VMEM((2 * _GSZ,), jnp.float32),
            pltpu.VMEM((_TSZ,), jnp.float32),
            pltpu.VMEM((chunk_px,), jnp.float32),
            pltpu.VMEM((_NIN * chunk_px,), jnp.float32),
            pltpu.VMEM((chunk_px,), jnp.float32),
            pltpu.VMEM((_NIN * chunk_px,), jnp.float32),
            pltpu.VMEM((_NOUT * chunk_px,), jnp.float32),
            pltpu.VMEM((_W,), jnp.int32),
            pltpu.VMEM((_W,), jnp.int32),
            pltpu.VMEM((_W,), jnp.float32),
            pltpu.SemaphoreType.DMA,
            pltpu.SemaphoreType.DMA,
            pltpu.SemaphoreType.DMA,
        ],
    )
    def sc_apply(grid_hbm, guide_hbm, image_hbm, x0_hbm, x1_hbm, dx_hbm,
                 out_hbm, grid2, t_buf, g_a, i_a, g_b, i_b, o_buf,
                 x0_buf, x1_buf, dx_buf, sem_a, sem_b, sem_g):
        wid = lax.axis_index("s") * 2 + lax.axis_index("c")

        def grid_copy(b, slot):
            return pltpu.make_async_copy(
                grid_hbm.at[b], grid2.at[pl.ds(slot * _GSZ, _GSZ)], sem_g)

        def in_copies(chunk, g_d, i_d, sem):
            b = chunk // 2
            h0 = wid * _RPW + (chunk % 2) * _RPC
            cps = [pltpu.make_async_copy(
                guide_hbm.at[b, pl.ds(h0 * _W, chunk_px)], g_d, sem)]
            for i in range(_NIN):
                cps.append(pltpu.make_async_copy(
                    image_hbm.at[b * _NIN + i, pl.ds(h0 * _W, chunk_px)],
                    i_d.at[pl.ds(i * chunk_px, chunk_px)], sem))
            return cps

        def issue(chunk, g_d, i_d, sem):
            for cp in in_copies(chunk, g_d, i_d, sem):
                cp.start()

        def drain(chunk, g_d, i_d, sem):
            for cp in in_copies(chunk, g_d, i_d, sem):
                cp.wait()

        def compute(chunk, g_buf, i_buf, pg):
            b = chunk // 2
            h0 = wid * _RPW + (chunk % 2) * _RPC
            for r in range(_RPC):
                h = h0 + r
                fy = lax.shift_right_arithmetic(h - _GH, 5)
                y0 = jnp.clip(fy, 0, _GH - 1) * _GW
                y1 = jnp.clip(fy + 1, 0, _GH - 1) * _GW
                hf = jnp.full((16,), h, dtype=jnp.int32).astype(jnp.float32)
                fyf = jnp.full((16,), fy, dtype=jnp.int32).astype(jnp.float32)
                dyv = (hf + np.float32(0.5)) * np.float32(1.0 / 32.0) \
                    - np.float32(0.5) - fyf

                def t_body(cz, c2):
                    src = pg + cz * (_GH * _GW)
                    g0 = grid2[pl.ds(src + y0, 16)]
                    g1 = grid2[pl.ds(src + y1, 16)]
                    t_buf[pl.ds(cz * _ZS, 16)] = g0 + dyv * (g1 - g0)
                    return c2

                lax.fori_loop(0, _C * _GD, t_body, 0)

                def vec_body(v, c2, r=r):
                    off = r * _W + v * 16
                    g = g_buf[pl.ds(off, 16)]
                    tz = g * np.float32(_GD) + np.float32(_GD - 0.5)
                    fzi = tz.astype(jnp.int32)
                    dz = tz - fzi.astype(jnp.float32)
                    z0 = jnp.clip(fzi - _GD, 0, _GD - 1) * _ZS
                    z1 = jnp.clip(fzi - (_GD - 1), 0, _GD - 1) * _ZS
                    x0 = x0_buf[pl.ds(v * 16, 16)]
                    x1 = x1_buf[pl.ds(v * 16, 16)]
                    dxv = dx_buf[pl.ds(v * 16, 16)]
                    i00 = x0 + z0
                    i01 = x1 + z0
                    i10 = x0 + z1
                    i11 = x1 + z1
                    ims = [i_buf[pl.ds(i * chunk_px + off, 16)]
                           for i in range(_NIN)]
                    for o in range(_NOUT):
                        acc = None
                        for j in range(_NIN + 1):
                            cb = (o * (_NIN + 1) + j) * _CS
                            a00 = plsc.load_gather(t_buf, [i00 + cb])
                            a01 = plsc.load_gather(t_buf, [i01 + cb])
                            a10 = plsc.load_gather(t_buf, [i10 + cb])
                            a11 = plsc.load_gather(t_buf, [i11 + cb])
                            a0 = a00 + dxv * (a01 - a00)
                            a1 = a10 + dxv * (a11 - a10)
                            cf = a0 + dz * (a1 - a0)
                            term = cf * ims[j] if j < _NIN else cf
                            acc = term if acc is None else acc + term
                        o_buf[pl.ds(o * chunk_px + off, 16)] = acc
                    return c2

                lax.fori_loop(0, _W // 16, vec_body, 0)

            for o in range(_NOUT):
                pltpu.sync_copy(
                    o_buf.at[pl.ds(o * chunk_px, chunk_px)],
                    out_hbm.at[b * _NOUT + o, pl.ds(h0 * _W, chunk_px)])

        pltpu.sync_copy(x0_hbm, x0_buf)
        pltpu.sync_copy(x1_hbm, x1_buf)
        pltpu.sync_copy(dx_hbm, dx_buf)
        grid_copy(0, 0).start()
        issue(0, g_a, i_a, sem_a)

        def outer(gi, carry):
            c0 = gi * 2
            c1 = c0 + 1
            p = lax.rem(gi, 2)
            pg = p * _GSZ
            issue(c1, g_b, i_b, sem_b)
            grid_copy(gi, p).wait()

            @pl.when(gi + 1 < _B)
            def _():
                grid_copy(gi + 1, 1 - p).start()

            drain(c0, g_a, i_a, sem_a)
            compute(c0, g_a, i_a, pg)

            @pl.when(c0 + 2 < n_chunks)
            def _():
                issue(c0 + 2, g_a, i_a, sem_a)

            drain(c1, g_b, i_b, sem_b)
            compute(c1, g_b, i_b, pg)
            return carry

        lax.fori_loop(0, n_chunks // 2, outer, 0)

    return sc_apply


_SC_APPLY = functools.lru_cache(maxsize=None)(_make_sc_kernel)
_X0, _X1, _DX = _axis_tables(_W, _GW)


@jax.jit
def kernel(grid, guide, image):
    out = _SC_APPLY()(
        grid.reshape(_B, _GSZ),
        guide.reshape(_B, _H * _W),
        image.reshape(_B * _NIN, _H * _W),
        jnp.asarray(_X0), jnp.asarray(_X1), jnp.asarray(_DX),
    )
    return out.reshape(_B, _NOUT, _H, _W)
